# baseline (speedup 1.0000x reference)
import jax
import jax.numpy as jnp
from jax.experimental import pallas as pl
from jax.experimental.pallas import tpu as pltpu

_B, _S, _D = 4, 8192, 768
_H = _D // 2
_E = 64
_T = _B * _S
_TILE = 512
_N1 = 3 * _H + 64
_INV_H = 1.0 / _H


def _router_body(x_ref, w1_ref, b1_ref, w2_ref, b2_ref,
                 wc2_ref, bc2_ref, rw_ref, conf_ref):
    x = x_ref[...]

    ha = jnp.dot(x, w1_ref[...], preferred_element_type=jnp.float32)
    ha = ha + b1_ref[...]

    c1 = jnp.tanh(ha[:, 3 * _H:])
    conf_lin = jnp.dot(c1, wc2_ref[...], preferred_element_type=jnp.float32)
    conf_ref[...] = jax.nn.sigmoid(conf_lin + bc2_ref[...])

    h = ha[:, :3 * _H]
    h = h * 0.5 * (1.0 + jax.lax.erf(h * 0.7071067811865476))

    logits = b2_ref[...]
    for lvl in range(3):
        hc = h[:, lvl * _H:(lvl + 1) * _H]
        s1 = jnp.sum(hc, axis=-1, keepdims=True)
        s2 = jnp.sum(hc * hc, axis=-1, keepdims=True)
        m = s1 * _INV_H
        v = s2 * _INV_H - m * m
        r = jax.lax.rsqrt(v + 1e-5)
        ll = jnp.dot((hc - m) * r, w2_ref[lvl * _H:(lvl + 1) * _H, :],
                     preferred_element_type=jnp.float32)
        logits = logits + ll

    m1 = jnp.max(logits, axis=-1, keepdims=True)
    m2 = jnp.max(jnp.where(logits < m1, logits, -jnp.inf),
                 axis=-1, keepdims=True)
    p = jnp.exp(logits - m1)
    s = jnp.sum(p, axis=-1, keepdims=True)
    pm = jnp.where(logits >= m2, p, 0.0)
    sm = jnp.sum(pm, axis=-1, keepdims=True)
    rw_ref[...] = pm / (sm + 1e-10 * s)


@jax.jit
def kernel(x, W1_0, b1_0, g_0, be_0, W2_0, b2_0,
           W1_1, b1_1, g_1, be_1, W2_1, b2_1,
           W1_2, b1_2, g_2, be_2, W2_2, b2_2,
           Wc1, bc1, Wc2, bc2):
    xt = x.reshape(_T, _D)

    w1 = jnp.concatenate([W1_0, W1_1, W1_2, Wc1], axis=1)
    b1 = jnp.concatenate([b1_0, b1_1, b1_2, bc1])[None, :]
    w2t = [W2_0, jnp.tile(W2_1, (1, 2)), jnp.tile(W2_2, (1, 4))]
    gs = [g_0, g_1, g_2]
    bes = [be_0, be_1, be_2]
    w2 = jnp.concatenate([g[:, None] * w for g, w in zip(gs, w2t)],
                         axis=0)
    b2 = (b2_0 + jnp.tile(b2_1, 2) + jnp.tile(b2_2, 4)
          + sum(be @ w for be, w in zip(bes, w2t)))[None, :]

    grid = (_T // _TILE,)
    full = lambda shape: pl.BlockSpec(shape, lambda i: (0,) * len(shape))
    rw, conf = pl.pallas_call(
        _router_body,
        grid=grid,
        in_specs=[
            pl.BlockSpec((_TILE, _D), lambda i: (i, 0)),
            full((_D, _N1)),
            full((1, _N1)),
            full((3 * _H, _E)),
            full((1, _E)),
            full((_E, 1)),
            full((1, 1)),
        ],
        out_specs=[
            pl.BlockSpec((_TILE, _E), lambda i: (i, 0)),
            pl.BlockSpec((_TILE, 1), lambda i: (i, 0)),
        ],
        out_shape=[
            jax.ShapeDtypeStruct((_T, _E), jnp.float32),
            jax.ShapeDtypeStruct((_T, 1), jnp.float32),
        ],
        compiler_params=pltpu.CompilerParams(
            dimension_semantics=("parallel",)),
    )(xt, w1, b1, w2, b2, Wc2, bc2[None, :])

    return rw.reshape(_B, _S, _E), conf.reshape(_B, _S, 1)


# speedup vs baseline: 1.1060x; 1.1060x over previous
import jax
import jax.numpy as jnp
from jax.experimental import pallas as pl
from jax.experimental.pallas import tpu as pltpu

_B, _S, _D = 4, 8192, 768
_H = _D // 2
_E = 64
_T = _B * _S
_TILE = 512
_INV_H = 1.0 / _H
_SQRT2 = 1.4142135623730951
_INV_SQRT2 = 0.7071067811865476


def _router_body(x_ref, w1_ref, b1_ref, w2_ref, b2_ref,
                 wc1_ref, bc1_ref, wc2_ref, bc2_ref, rw_ref, conf_ref):
    x = x_ref[...]

    c1 = jnp.tanh(
        jnp.dot(x, wc1_ref[...], preferred_element_type=jnp.float32)
        + bc1_ref[...])
    conf_lin = jnp.sum(c1 * wc2_ref[...], axis=-1, keepdims=True) + bc2_ref[...]
    conf_ref[...] = jax.nn.sigmoid(conf_lin)

    hs = jnp.dot(x, w1_ref[...], preferred_element_type=jnp.float32)
    hs = hs + b1_ref[...]
    h = (_INV_SQRT2 * hs) * (1.0 + jax.lax.erf(hs))

    logits = b2_ref[...]
    for lvl in range(3):
        hc = h[:, lvl * _H:(lvl + 1) * _H]
        s1 = jnp.sum(hc, axis=-1, keepdims=True)
        s2 = jnp.sum(hc * hc, axis=-1, keepdims=True)
        m = s1 * _INV_H
        v = s2 * _INV_H - m * m
        r = jax.lax.rsqrt(v + 1e-5)
        ll = jnp.dot((hc - m) * r, w2_ref[lvl * _H:(lvl + 1) * _H, :],
                     preferred_element_type=jnp.float32)
        logits = logits + ll

    m1 = jnp.max(logits, axis=-1, keepdims=True)
    m2 = jnp.max(jnp.where(logits < m1, logits, -jnp.inf),
                 axis=-1, keepdims=True)
    p = jnp.exp(logits - m1)
    s = jnp.sum(p, axis=-1, keepdims=True)
    pm = jnp.where(logits >= m2, p, 0.0)
    sm = jnp.sum(pm, axis=-1, keepdims=True)
    rw_ref[...] = pm / (sm + 1e-10 * s)


@jax.jit
def kernel(x, W1_0, b1_0, g_0, be_0, W2_0, b2_0,
           W1_1, b1_1, g_1, be_1, W2_1, b2_1,
           W1_2, b1_2, g_2, be_2, W2_2, b2_2,
           Wc1, bc1, Wc2, bc2):
    xt = x.reshape(_T, _D)

    w1 = jnp.concatenate([W1_0, W1_1, W1_2], axis=1) * _INV_SQRT2
    b1 = (jnp.concatenate([b1_0, b1_1, b1_2]) * _INV_SQRT2)[None, :]
    w2t = [W2_0, jnp.tile(W2_1, (1, 2)), jnp.tile(W2_2, (1, 4))]
    gs = [g_0, g_1, g_2]
    bes = [be_0, be_1, be_2]
    w2 = jnp.concatenate([g[:, None] * w for g, w in zip(gs, w2t)],
                         axis=0)
    b2 = (b2_0 + jnp.tile(b2_1, 2) + jnp.tile(b2_2, 4)
          + sum(be @ w for be, w in zip(bes, w2t)))[None, :]

    grid = (_T // _TILE,)
    full = lambda shape: pl.BlockSpec(shape, lambda i: (0,) * len(shape))
    rw, conf = pl.pallas_call(
        _router_body,
        grid=grid,
        in_specs=[
            pl.BlockSpec((_TILE, _D), lambda i: (i, 0)),
            full((_D, 3 * _H)),
            full((1, 3 * _H)),
            full((3 * _H, _E)),
            full((1, _E)),
            full((_D, _E)),
            full((1, _E)),
            full((1, _E)),
            full((1, 1)),
        ],
        out_specs=[
            pl.BlockSpec((_TILE, _E), lambda i: (i, 0)),
            pl.BlockSpec((_TILE, 1), lambda i: (i, 0)),
        ],
        out_shape=[
            jax.ShapeDtypeStruct((_T, _E), jnp.float32),
            jax.ShapeDtypeStruct((_T, 1), jnp.float32),
        ],
        compiler_params=pltpu.CompilerParams(
            dimension_semantics=("parallel",)),
    )(xt, w1, b1, w2, b2, Wc1, bc1[None, :], Wc2.reshape(1, _E),
      bc2[None, :])

    return rw.reshape(_B, _S, _E), conf.reshape(_B, _S, 1)


# speedup vs baseline: 1.1396x; 1.0303x over previous
import jax
import jax.numpy as jnp
from jax.experimental import pallas as pl
from jax.experimental.pallas import tpu as pltpu

_B, _S, _D = 4, 8192, 768
_H = _D // 2
_E = 64
_T = _B * _S
_TILE = 1024
_INV_H = 1.0 / _H
_SQRT2 = 1.4142135623730951
_INV_SQRT2 = 0.7071067811865476


def _router_body(x_ref, w1_ref, b1_ref, w2_ref, b2_ref,
                 wc1_ref, bc1_ref, wc2_ref, bc2_ref, rw_ref, conf_ref):
    x = x_ref[...]

    c1 = jnp.tanh(
        jnp.dot(x, wc1_ref[...], preferred_element_type=jnp.float32)
        + bc1_ref[...])
    conf_lin = jnp.sum(c1 * wc2_ref[...], axis=-1, keepdims=True) + bc2_ref[...]
    conf_ref[...] = jax.nn.sigmoid(conf_lin)

    h = jnp.dot(x, w1_ref[...], preferred_element_type=jnp.float32)
    h = h + b1_ref[...]
    h = (0.5 * h) * (1.0 + jax.lax.erf(h * _INV_SQRT2))

    logits = b2_ref[...]
    for lvl in range(3):
        hc = h[:, lvl * _H:(lvl + 1) * _H]
        s1 = jnp.sum(hc, axis=-1, keepdims=True)
        s2 = jnp.sum(hc * hc, axis=-1, keepdims=True)
        m = s1 * _INV_H
        v = s2 * _INV_H - m * m
        r = jax.lax.rsqrt(v + 1e-5)
        ll = jnp.dot((hc - m) * r, w2_ref[lvl * _H:(lvl + 1) * _H, :],
                     preferred_element_type=jnp.float32)
        logits = logits + ll

    m1 = jnp.max(logits, axis=-1, keepdims=True)
    m2 = jnp.max(jnp.where(logits < m1, logits, -jnp.inf),
                 axis=-1, keepdims=True)
    p = jnp.exp(logits - m1)
    s = jnp.sum(p, axis=-1, keepdims=True)
    pm = jnp.where(logits >= m2, p, 0.0)
    sm = jnp.sum(pm, axis=-1, keepdims=True)
    rw_ref[...] = pm / (sm + 1e-10 * s)


@jax.jit
def kernel(x, W1_0, b1_0, g_0, be_0, W2_0, b2_0,
           W1_1, b1_1, g_1, be_1, W2_1, b2_1,
           W1_2, b1_2, g_2, be_2, W2_2, b2_2,
           Wc1, bc1, Wc2, bc2):
    xt = x.reshape(_T, _D)

    w1 = jnp.concatenate([W1_0, W1_1, W1_2], axis=1)
    b1 = jnp.concatenate([b1_0, b1_1, b1_2])[None, :]
    w2t = [W2_0, jnp.tile(W2_1, (1, 2)), jnp.tile(W2_2, (1, 4))]
    gs = [g_0, g_1, g_2]
    bes = [be_0, be_1, be_2]
    w2 = jnp.concatenate([g[:, None] * w for g, w in zip(gs, w2t)],
                         axis=0)
    b2 = (b2_0 + jnp.tile(b2_1, 2) + jnp.tile(b2_2, 4)
          + sum(be @ w for be, w in zip(bes, w2t)))[None, :]

    grid = (_T // _TILE,)
    full = lambda shape: pl.BlockSpec(shape, lambda i: (0,) * len(shape))
    rw, conf = pl.pallas_call(
        _router_body,
        grid=grid,
        in_specs=[
            pl.BlockSpec((_TILE, _D), lambda i: (i, 0)),
            full((_D, 3 * _H)),
            full((1, 3 * _H)),
            full((3 * _H, _E)),
            full((1, _E)),
            full((_D, _E)),
            full((1, _E)),
            full((1, _E)),
            full((1, 1)),
        ],
        out_specs=[
            pl.BlockSpec((_TILE, _E), lambda i: (i, 0)),
            pl.BlockSpec((_TILE, 1), lambda i: (i, 0)),
        ],
        out_shape=[
            jax.ShapeDtypeStruct((_T, _E), jnp.float32),
            jax.ShapeDtypeStruct((_T, 1), jnp.float32),
        ],
        compiler_params=pltpu.CompilerParams(
            dimension_semantics=("parallel",)),
    )(xt, w1, b1, w2, b2, Wc1, bc1[None, :], Wc2.reshape(1, _E),
      bc2[None, :])

    return rw.reshape(_B, _S, _E), conf.reshape(_B, _S, 1)
